# Initial kernel scaffold; baseline (speedup 1.0000x reference)
import jax
import jax.numpy as jnp
from jax.experimental import pallas as pl


def kernel(indices, table):
    raise NotImplementedError("write your pallas kernel here")


# speedup vs baseline: 1.1029x; 1.1029x over previous
import functools

import jax
import jax.numpy as jnp
from jax import lax
from jax.experimental import pallas as pl
from jax.experimental.pallas import tpu as pltpu
from jax.experimental.pallas import tpu_sc as plsc

_NUM_CORES = 2
_NUM_SUBCORES = 16
_NUM_WORKERS = _NUM_CORES * _NUM_SUBCORES


def _gather_kernel(n_rows, embed_dim, chunk, idx_hbm, table_hbm, out_hbm,
                   idx_v, rows_v, sem):
  wid = lax.axis_index("s") * _NUM_CORES + lax.axis_index("c")
  rows_per_w = n_rows // _NUM_WORKERS
  n_chunks = rows_per_w // chunk
  base = wid * rows_per_w

  def step(i, carry):
    off = base + i * chunk
    pltpu.sync_copy(idx_hbm.at[pl.ds(off, chunk)], idx_v)
    pltpu.async_copy(table_hbm.at[idx_v], rows_v, sem).wait()
    pltpu.sync_copy(rows_v, out_hbm.at[pl.ds(off, chunk)])
    return carry

  lax.fori_loop(0, n_chunks, step, 0)


def kernel(indices, table):
  batch, hist = indices.shape
  n_rows = batch * hist
  embed_dim = table.shape[1]
  chunk = 1600

  flat_idx = indices.reshape(n_rows).astype(jnp.int32)

  mesh = plsc.VectorSubcoreMesh(core_axis_name="c", subcore_axis_name="s")
  k = pl.kernel(
      functools.partial(_gather_kernel, n_rows, embed_dim, chunk),
      out_type=jax.ShapeDtypeStruct((n_rows, embed_dim), jnp.float32),
      mesh=mesh,
      scratch_types=[
          pltpu.VMEM((chunk,), jnp.int32),
          pltpu.VMEM((chunk, embed_dim), jnp.float32),
          pltpu.SemaphoreType.DMA,
      ],
      compiler_params=pltpu.CompilerParams(use_tc_tiling_on_sc=False),
  )
  out = k(flat_idx, table)
  return out.reshape(batch, hist, embed_dim)


# speedup vs baseline: 1.1088x; 1.0054x over previous
import functools

import jax
import jax.numpy as jnp
from jax import lax
from jax.experimental import pallas as pl
from jax.experimental.pallas import tpu as pltpu
from jax.experimental.pallas import tpu_sc as plsc

_NUM_CORES = 2
_NUM_SUBCORES = 16
_NUM_WORKERS = _NUM_CORES * _NUM_SUBCORES
_NBUF = 2


def _gather_kernel(n_rows, chunk, idx_hbm, table_hbm, out_hbm,
                   idx_v0, idx_v1, rows_v0, rows_v1,
                   isem, gsem, wsem):
  wid = lax.axis_index("s") * _NUM_CORES + lax.axis_index("c")
  rows_per_w = n_rows // _NUM_WORKERS
  n_chunks = rows_per_w // chunk
  base = wid * rows_per_w
  idx_v = (idx_v0, idx_v1)
  rows_v = (rows_v0, rows_v1)

  def off(i):
    return base + i * chunk

  def start_idx(i):
    s = i % _NBUF
    pltpu.async_copy(idx_hbm.at[pl.ds(off(i), chunk)], idx_v[s],
                     isem.at[s])

  def wait_idx(i):
    s = i % _NBUF
    pltpu.make_async_copy(idx_hbm.at[pl.ds(off(i), chunk)], idx_v[s],
                          isem.at[s]).wait()

  def start_gather(i):
    s = i % _NBUF
    pltpu.async_copy(table_hbm.at[idx_v[s]], rows_v[s], gsem.at[s])

  def wait_gather(i):
    s = i % _NBUF
    pltpu.make_async_copy(table_hbm.at[idx_v[s]], rows_v[s],
                          gsem.at[s]).wait()

  def start_wb(i):
    s = i % _NBUF
    pltpu.async_copy(rows_v[s], out_hbm.at[pl.ds(off(i), chunk)],
                     wsem.at[s])

  def wait_wb(i):
    s = i % _NBUF
    pltpu.make_async_copy(rows_v[s], out_hbm.at[pl.ds(off(i), chunk)],
                          wsem.at[s]).wait()

  for i in range(min(_NBUF, n_chunks)):
    start_idx(i)
  for i in range(n_chunks):
    wait_idx(i)
    if i >= _NBUF:
      wait_wb(i - _NBUF)
    start_gather(i)
    wait_gather(i)
    if i + _NBUF < n_chunks:
      start_idx(i + _NBUF)
    start_wb(i)
  for i in range(max(0, n_chunks - _NBUF), n_chunks):
    wait_wb(i)


def kernel(indices, table):
  batch, hist = indices.shape
  n_rows = batch * hist
  embed_dim = table.shape[1]
  chunk = 1600

  flat_idx = indices.reshape(n_rows).astype(jnp.int32)

  mesh = plsc.VectorSubcoreMesh(core_axis_name="c", subcore_axis_name="s")
  k = pl.kernel(
      functools.partial(_gather_kernel, n_rows, chunk),
      out_type=jax.ShapeDtypeStruct((n_rows, embed_dim), jnp.float32),
      mesh=mesh,
      scratch_types=[
          pltpu.VMEM((chunk,), jnp.int32),
          pltpu.VMEM((chunk,), jnp.int32),
          pltpu.VMEM((chunk, embed_dim), jnp.float32),
          pltpu.VMEM((chunk, embed_dim), jnp.float32),
          pltpu.SemaphoreType.DMA((_NBUF,)),
          pltpu.SemaphoreType.DMA((_NBUF,)),
          pltpu.SemaphoreType.DMA((_NBUF,)),
      ],
      compiler_params=pltpu.CompilerParams(use_tc_tiling_on_sc=False),
  )
  out = k(flat_idx, table)
  return out.reshape(batch, hist, embed_dim)


# speedup vs baseline: 1.7847x; 1.6095x over previous
import functools

import jax
import jax.numpy as jnp
from jax import lax
from jax.experimental import pallas as pl
from jax.experimental.pallas import tpu as pltpu
from jax.experimental.pallas import tpu_sc as plsc

_NUM_CORES = 2
_NUM_SUBCORES = 16
_NUM_WORKERS = _NUM_CORES * _NUM_SUBCORES
_NBUF = 2


def _gather_kernel(n_rows, hist, chunk, idx_hbm, table_hbm, out_hbm,
                   idx_v0, idx_v1, rows_v0, rows_v1,
                   isem, gsem, wsem):
  wid = lax.axis_index("s") * _NUM_CORES + lax.axis_index("c")
  rows_per_w = n_rows // _NUM_WORKERS
  n_chunks = rows_per_w // chunk
  b_per_chunk = chunk // hist
  base = wid * rows_per_w
  b_base = wid * (rows_per_w // hist)
  idx_v = (idx_v0, idx_v1)
  rows_v = (rows_v0, rows_v1)

  def off(i):
    return base + i * chunk

  def start_idx(i):
    s = i % _NBUF
    pltpu.async_copy(idx_hbm.at[pl.ds(off(i), chunk)], idx_v[s],
                     isem.at[s])

  def wait_idx(i):
    s = i % _NBUF
    pltpu.make_async_copy(idx_hbm.at[pl.ds(off(i), chunk)], idx_v[s],
                          isem.at[s]).wait()

  def start_gather(i):
    s = i % _NBUF
    pltpu.async_copy(table_hbm.at[idx_v[s]], rows_v[s], gsem.at[s])

  def wait_gather(i):
    s = i % _NBUF
    pltpu.make_async_copy(table_hbm.at[idx_v[s]], rows_v[s],
                          gsem.at[s]).wait()

  def start_wb(i):
    s = i % _NBUF
    b0 = b_base + i * b_per_chunk
    for j in range(b_per_chunk):
      pltpu.async_copy(rows_v[s].at[pl.ds(j * hist, hist)],
                       out_hbm.at[b0 + j], wsem.at[s])

  def wait_wb(i):
    s = i % _NBUF
    b0 = b_base + i * b_per_chunk
    for j in range(b_per_chunk):
      pltpu.make_async_copy(rows_v[s].at[pl.ds(j * hist, hist)],
                            out_hbm.at[b0 + j], wsem.at[s]).wait()

  for i in range(min(_NBUF, n_chunks)):
    start_idx(i)
  for i in range(n_chunks):
    wait_idx(i)
    if i >= _NBUF:
      wait_wb(i - _NBUF)
    start_gather(i)
    wait_gather(i)
    if i + _NBUF < n_chunks:
      start_idx(i + _NBUF)
    start_wb(i)
  for i in range(max(0, n_chunks - _NBUF), n_chunks):
    wait_wb(i)


def kernel(indices, table):
  batch, hist = indices.shape
  n_rows = batch * hist
  embed_dim = table.shape[1]
  chunk = 1600

  flat_idx = indices.reshape(n_rows).astype(jnp.int32)

  mesh = plsc.VectorSubcoreMesh(core_axis_name="c", subcore_axis_name="s")
  k = pl.kernel(
      functools.partial(_gather_kernel, n_rows, hist, chunk),
      out_type=jax.ShapeDtypeStruct((batch, hist, embed_dim), jnp.float32),
      mesh=mesh,
      scratch_types=[
          pltpu.VMEM((chunk,), jnp.int32),
          pltpu.VMEM((chunk,), jnp.int32),
          pltpu.VMEM((chunk, embed_dim), jnp.float32),
          pltpu.VMEM((chunk, embed_dim), jnp.float32),
          pltpu.SemaphoreType.DMA((_NBUF,)),
          pltpu.SemaphoreType.DMA((_NBUF,)),
          pltpu.SemaphoreType.DMA((_NBUF,)),
      ],
      compiler_params=pltpu.CompilerParams(use_tc_tiling_on_sc=False),
  )
  return k(flat_idx, table)
